# baseline (speedup 1.0000x reference)
import jax
import jax.numpy as jnp
from jax.experimental import pallas as pl

N = 10000
X_DIM = 128
H_DIM = 128
G_DIM = 128
BLK = 5120


def _fused_mlp_kernel(x_ref, t_ref, wphi_ref, bphi_ref, wg_ref, bg_ref,
                      wg2_ref, bg2_ref, wt01_ref, bt01_ref, wt11_ref,
                      bt11_ref, y1_ref, y0_ref, phi_ref):
    bf = jnp.bfloat16
    x = x_ref[...].astype(bf)
    phi = jnp.dot(x, wphi_ref[...].astype(bf),
                  preferred_element_type=jnp.float32)
    phi = phi + bphi_ref[...]
    phi_ref[...] = phi

    t_col = t_ref[...].reshape(BLK, 1)
    h = t_col * phi
    h = jnp.dot(h.astype(bf), wg_ref[...].astype(bf),
                preferred_element_type=jnp.float32) + bg_ref[...]
    h = jnp.maximum(h, 0.0)
    h = jnp.dot(h.astype(bf), wg2_ref[...].astype(bf),
                preferred_element_type=jnp.float32) + bg2_ref[...]
    h = jnp.maximum(h, 0.0)

    hb = h.astype(bf)
    w01g = wt01_ref[pl.ds(H_DIM, G_DIM), :].astype(bf)
    w11p = wt11_ref[pl.ds(0, H_DIM), :].astype(bf)
    w11g = wt11_ref[pl.ds(H_DIM, G_DIM), :].astype(bf)
    y0 = jnp.dot(hb, w01g, preferred_element_type=jnp.float32)
    y1 = (jnp.dot(phi.astype(bf), w11p, preferred_element_type=jnp.float32)
          + jnp.dot(hb, w11g, preferred_element_type=jnp.float32))
    y0_ref[...] = y0.reshape(BLK) + bt01_ref[...]
    y1_ref[...] = y1.reshape(BLK) + bt11_ref[...]


def kernel(features, treatments, edge_index, W_phi, b_phi, W_g, b_g,
           W_g2, b_g2, W_t01, b_t01, W_t11, b_t11):
    del edge_index

    grid = ((N + BLK - 1) // BLK,)
    row_spec = pl.BlockSpec((BLK, X_DIM), lambda i: (i, 0))

    def full(shape):
        return pl.BlockSpec(shape, lambda i: (0,) * len(shape))

    y1, y0, phi_x = pl.pallas_call(
        _fused_mlp_kernel,
        grid=grid,
        in_specs=[
            row_spec,
            pl.BlockSpec((BLK,), lambda i: (i,)),
            full((X_DIM, H_DIM)),
            full((H_DIM,)),
            full((H_DIM, G_DIM)),
            full((G_DIM,)),
            full((G_DIM, G_DIM)),
            full((G_DIM,)),
            full((H_DIM + G_DIM, 1)),
            full((1,)),
            full((H_DIM + G_DIM, 1)),
            full((1,)),
        ],
        out_specs=[pl.BlockSpec((BLK,), lambda i: (i,)),
                   pl.BlockSpec((BLK,), lambda i: (i,)), row_spec],
        out_shape=[
            jax.ShapeDtypeStruct((N,), jnp.float32),
            jax.ShapeDtypeStruct((N,), jnp.float32),
            jax.ShapeDtypeStruct((N, H_DIM), jnp.float32),
        ],
    )(features, treatments, W_phi, b_phi, W_g, b_g, W_g2, b_g2,
      W_t01, b_t01, W_t11, b_t11)

    return (y1, y0, phi_x)


# speedup vs baseline: 1.6610x; 1.6610x over previous
import jax
import jax.numpy as jnp
from jax.experimental import pallas as pl

N = 10000
X_DIM = 128
H_DIM = 128
G_DIM = 128
BLK = 5120


def _fused_mlp_kernel(x_ref, t_ref, wphi_ref, bphi_ref, wg_ref, bg_ref,
                      wg2_ref, bg2_ref, wt01_ref, bt01_ref, wt11_ref,
                      bt11_ref, y1_ref, y0_ref, phi_ref):
    bf = jnp.bfloat16
    x = x_ref[...].astype(bf)
    phi = jnp.dot(x, wphi_ref[...].astype(bf),
                  preferred_element_type=jnp.float32)
    phi = phi + bphi_ref[...]
    phi_ref[...] = phi

    t_col = t_ref[...].reshape(BLK, 1)
    h = t_col * phi
    h = jnp.dot(h.astype(bf), wg_ref[...].astype(bf),
                preferred_element_type=jnp.float32) + bg_ref[...]
    h = jnp.maximum(h, 0.0)
    h = jnp.dot(h.astype(bf), wg2_ref[...].astype(bf),
                preferred_element_type=jnp.float32) + bg2_ref[...]
    h = jnp.maximum(h, 0.0)

    hb = h.astype(bf)
    w01g = wt01_ref[pl.ds(H_DIM, G_DIM), :].reshape(1, G_DIM)
    w11p = wt11_ref[pl.ds(0, H_DIM), :].reshape(1, H_DIM)
    w11g = wt11_ref[pl.ds(H_DIM, G_DIM), :].reshape(1, G_DIM)
    wh2 = jnp.concatenate([w01g, w11g], axis=0).astype(bf)
    dn = (((1,), (1,)), ((), ()))
    yh = jax.lax.dot_general(wh2, hb, dn,
                             preferred_element_type=jnp.float32)
    yp = jax.lax.dot_general(w11p.astype(bf), phi.astype(bf), dn,
                             preferred_element_type=jnp.float32)
    y0_ref[...] = yh[0] + bt01_ref[...]
    y1_ref[...] = yh[1] + yp[0] + bt11_ref[...]


def kernel(features, treatments, edge_index, W_phi, b_phi, W_g, b_g,
           W_g2, b_g2, W_t01, b_t01, W_t11, b_t11):
    del edge_index

    grid = ((N + BLK - 1) // BLK,)
    row_spec = pl.BlockSpec((BLK, X_DIM), lambda i: (i, 0))

    def full(shape):
        return pl.BlockSpec(shape, lambda i: (0,) * len(shape))

    y1, y0, phi_x = pl.pallas_call(
        _fused_mlp_kernel,
        grid=grid,
        in_specs=[
            row_spec,
            pl.BlockSpec((BLK,), lambda i: (i,)),
            full((X_DIM, H_DIM)),
            full((H_DIM,)),
            full((H_DIM, G_DIM)),
            full((G_DIM,)),
            full((G_DIM, G_DIM)),
            full((G_DIM,)),
            full((H_DIM + G_DIM, 1)),
            full((1,)),
            full((H_DIM + G_DIM, 1)),
            full((1,)),
        ],
        out_specs=[pl.BlockSpec((BLK,), lambda i: (i,)),
                   pl.BlockSpec((BLK,), lambda i: (i,)), row_spec],
        out_shape=[
            jax.ShapeDtypeStruct((N,), jnp.float32),
            jax.ShapeDtypeStruct((N,), jnp.float32),
            jax.ShapeDtypeStruct((N, H_DIM), jnp.float32),
        ],
    )(features, treatments, W_phi, b_phi, W_g, b_g, W_g2, b_g2,
      W_t01, b_t01, W_t11, b_t11)

    return (y1, y0, phi_x)
